# Initial kernel scaffold; baseline (speedup 1.0000x reference)
import jax
import jax.numpy as jnp
from jax.experimental import pallas as pl


def kernel(chars, table, W, b):
    raise NotImplementedError("write your pallas kernel here")


# speedup vs baseline: 1.7309x; 1.7309x over previous
import jax
import jax.numpy as jnp
from jax.experimental import pallas as pl

N = 68
VOCAB = 101
EMB = 32


def _fused_kernel(chars_ref, table_ref, w_ref, b_ref, emb_ref, loss_ref):
    c = chars_ref[0, :]
    vocab_ids = jax.lax.broadcasted_iota(jnp.int32, (N, VOCAB), 1)
    onehot = (c[:, None] == vocab_ids).astype(jnp.float32)
    emb0 = jnp.dot(onehot, table_ref[...], preferred_element_type=jnp.float32)
    h = jnp.maximum(emb0, 0.0)
    emb = jax.lax.dot_general(h, w_ref[...], (((1,), (1,)), ((), ())),
                              preferred_element_type=jnp.float32)
    emb = emb + b_ref[0, :][None, :]
    emb_ref[...] = emb

    rows = jax.lax.broadcasted_iota(jnp.int32, (N, EMB), 0)

    def seg_mean(lo, hi):
        m = ((rows >= lo) & (rows < hi)).astype(jnp.float32)
        return jnp.sum(emb * m, axis=0, keepdims=True) / (hi - lo)

    def seg_std_sum(lo, hi):
        n = hi - lo
        m = ((rows >= lo) & (rows < hi)).astype(jnp.float32)
        mean = jnp.sum(emb * m, axis=0, keepdims=True) / n
        var = jnp.sum(m * (emb - mean) ** 2, axis=0, keepdims=True) / (n - 1)
        return jnp.sum(jnp.sqrt(var))

    nr = seg_mean(0, 10)
    ar = seg_mean(10, 36)
    sr = seg_mean(36, N)
    loss = seg_std_sum(0, 10) + seg_std_sum(10, 26) + seg_std_sum(36, N)
    rd = (nr - ar) ** 2 + (sr - ar) ** 2 + (nr - sr) ** 2
    loss = loss + 1.0 / jnp.sum(rd)
    loss_ref[...] = jnp.reshape(loss, (1, 1))


def kernel(chars, table, W, b):
    chars2 = chars.astype(jnp.int32).reshape(1, N)
    b2 = b.reshape(1, EMB)
    emb, loss = pl.pallas_call(
        _fused_kernel,
        out_shape=(
            jax.ShapeDtypeStruct((N, EMB), jnp.float32),
            jax.ShapeDtypeStruct((1, 1), jnp.float32),
        ),
    )(chars2, table, W, b2)
    return (loss[0, 0], emb)
